# Initial kernel scaffold; baseline (speedup 1.0000x reference)
import jax
import jax.numpy as jnp
from jax.experimental import pallas as pl


def kernel(user_id, item_id, user_factors, item_factors, user_bias, item_bias, global_bias):
    raise NotImplementedError("write your pallas kernel here")


# speedup vs baseline: 1.2530x; 1.2530x over previous
import jax
import jax.numpy as jnp
from jax import lax
from jax.experimental import pallas as pl
from jax.experimental.pallas import tpu as pltpu
from jax.experimental.pallas import tpu_sc as plsc

NUM_ROWS = 100000
EMBED = 64
PACK = 128
BATCH = 16384

NC = 2
NS = 16
L = 16
NW = NC * NS
B_PER_W = BATCH // NW
CHUNK = 256

TC_BC = 4096

_SHUF_DNUMS = lax.GatherDimensionNumbers(
    offset_dims=(), collapsed_slice_dims=(0,), start_index_map=(0,))


def _shuffle(x, idx):
    return lax.gather(x, idx[:, None], _SHUF_DNUMS, (1,),
                      mode=lax.GatherScatterMode.PROMISE_IN_BOUNDS)


def _tp_body(u_ref, v_ref, o_ref):
    o_ref[:, pl.ds(0, EMBED)] = u_ref[...].T
    o_ref[:, pl.ds(EMBED, EMBED)] = v_ref[...].T


_tp_call = pl.pallas_call(
    _tp_body,
    grid=(pl.cdiv(NUM_ROWS, TC_BC),),
    in_specs=[
        pl.BlockSpec((EMBED, TC_BC), lambda i: (0, i)),
        pl.BlockSpec((EMBED, TC_BC), lambda i: (0, i)),
    ],
    out_specs=pl.BlockSpec((TC_BC, PACK), lambda i: (i, 0)),
    out_shape=jax.ShapeDtypeStruct((NUM_ROWS, PACK), jnp.float32),
)


def _fm_body(uid_hbm, iid_hbm, w_hbm, ub_hbm, ib_hbm, gb_hbm,
             pred_hbm, ctr_hbm,
             idx_u, idx_v, u_rows, v_rows, ub_v, ib_v, gb_v,
             pred_v, ctr_v, sem):
    wid = lax.axis_index("s") * NC + lax.axis_index("c")
    base = wid * B_PER_W

    pltpu.sync_copy(uid_hbm.at[pl.ds(base, B_PER_W)], idx_u)
    pltpu.sync_copy(iid_hbm.at[pl.ds(base, B_PER_W)], idx_v)
    cb1 = pltpu.async_copy(ub_hbm.at[idx_u], ub_v, sem)
    cb2 = pltpu.async_copy(ib_hbm.at[idx_v], ib_v, sem)
    pltpu.sync_copy(gb_hbm, gb_v)
    cb1.wait()
    cb2.wait()

    gb = gb_v[...]
    lane = lax.iota(jnp.int32, L)

    def chunk_loop(half, carry):
        cbase = half * CHUNK
        c1 = pltpu.async_copy(w_hbm.at[idx_u.at[pl.ds(cbase, CHUNK)]],
                              u_rows, sem)
        c2 = pltpu.async_copy(w_hbm.at[idx_v.at[pl.ds(cbase, CHUNK)]],
                              v_rows, sem)
        c1.wait()
        c2.wait()

        def body(g, carry2):
            res = jnp.zeros((L,), jnp.float32)
            for j in range(L):
                r = g * L + j
                acc = None
                for c in range(EMBED // L):
                    uu = u_rows[r, pl.ds(c * L, L)]
                    vv = v_rows[r, pl.ds(EMBED + c * L, L)]
                    acc = uu * vv if acc is None else acc + uu * vv
                s = acc
                for k in (8, 4, 2, 1):
                    s = s + _shuffle(s, lane ^ k)
                res = jnp.where(lane == j, s, res)
            off = cbase + g * L
            pred16 = res + ub_v[pl.ds(off, L)] + ib_v[pl.ds(off, L)] + gb
            pred_v[pl.ds(off, L)] = pred16
            ctr_v[pl.ds(off, L)] = 1.0 / (1.0 + jnp.exp(-pred16))
            return carry2

        lax.fori_loop(0, CHUNK // L, body, 0)
        return carry

    lax.fori_loop(0, B_PER_W // CHUNK, chunk_loop, 0)

    pltpu.sync_copy(pred_v, pred_hbm.at[pl.ds(base, B_PER_W)])
    pltpu.sync_copy(ctr_v, ctr_hbm.at[pl.ds(base, B_PER_W)])


_fm_call = pl.kernel(
    _fm_body,
    out_type=(
        jax.ShapeDtypeStruct((BATCH,), jnp.float32),
        jax.ShapeDtypeStruct((BATCH,), jnp.float32),
    ),
    mesh=plsc.VectorSubcoreMesh(
        core_axis_name="c", subcore_axis_name="s",
        num_cores=NC, num_subcores=NS,
    ),
    scratch_types=[
        pltpu.VMEM((B_PER_W,), jnp.int32),
        pltpu.VMEM((B_PER_W,), jnp.int32),
        pltpu.VMEM((CHUNK, PACK), jnp.float32),
        pltpu.VMEM((CHUNK, PACK), jnp.float32),
        pltpu.VMEM((B_PER_W,), jnp.float32),
        pltpu.VMEM((B_PER_W,), jnp.float32),
        pltpu.VMEM((L,), jnp.float32),
        pltpu.VMEM((B_PER_W,), jnp.float32),
        pltpu.VMEM((B_PER_W,), jnp.float32),
        pltpu.SemaphoreType.DMA,
    ],
)


@jax.jit
def kernel(user_id, item_id, user_factors, item_factors, user_bias,
           item_bias, global_bias):
    fused = _tp_call(user_factors.T, item_factors.T)
    gb16 = jnp.broadcast_to(global_bias.astype(jnp.float32), (L,))
    pred, ctr = _fm_call(user_id, item_id, fused,
                         user_bias, item_bias, gb16)
    return pred, ctr


# speedup vs baseline: 1.3531x; 1.0799x over previous
import jax
import jax.numpy as jnp
from jax import lax
from jax.experimental import pallas as pl
from jax.experimental.pallas import tpu as pltpu
from jax.experimental.pallas import tpu_sc as plsc

NUM_ROWS = 100000
EMBED = 64
PACK = 128
BATCH = 16384

NC = 2
NS = 16
L = 16
NW = NC * NS
B_PER_W = BATCH // NW
CHUNK = 128
NCHUNK = B_PER_W // CHUNK

TC_BC = 8192

_SHUF_DNUMS = lax.GatherDimensionNumbers(
    offset_dims=(), collapsed_slice_dims=(0,), start_index_map=(0,))


def _shuffle(x, idx):
    return lax.gather(x, idx[:, None], _SHUF_DNUMS, (1,),
                      mode=lax.GatherScatterMode.PROMISE_IN_BOUNDS)


def _tp_body(u_ref, v_ref, o_ref):
    o_ref[...] = jnp.concatenate(
        [u_ref[...].T, v_ref[...].T], axis=1)


_tp_call = pl.pallas_call(
    _tp_body,
    grid=(pl.cdiv(NUM_ROWS, TC_BC),),
    in_specs=[
        pl.BlockSpec((EMBED, TC_BC), lambda i: (0, i)),
        pl.BlockSpec((EMBED, TC_BC), lambda i: (0, i)),
    ],
    out_specs=pl.BlockSpec((TC_BC, PACK), lambda i: (i, 0)),
    out_shape=jax.ShapeDtypeStruct((NUM_ROWS, PACK), jnp.float32),
)


def _fm_body(uid_hbm, iid_hbm, w_hbm, ub_hbm, ib_hbm, gb_hbm,
             pred_hbm, ctr_hbm,
             idx_u, idx_v, u_a, u_b, v_a, v_b, ub_v, ib_v, gb_v,
             pred_v, ctr_v, sem_bias, sem_a, sem_b):
    wid = lax.axis_index("s") * NC + lax.axis_index("c")
    base = wid * B_PER_W

    ubufs = (u_a, u_b)
    vbufs = (v_a, v_b)
    sems = (sem_a, sem_b)

    pltpu.sync_copy(uid_hbm.at[pl.ds(base, B_PER_W)], idx_u)
    pltpu.sync_copy(iid_hbm.at[pl.ds(base, B_PER_W)], idx_v)
    cb1 = pltpu.async_copy(ub_hbm.at[idx_u], ub_v, sem_bias)
    cb2 = pltpu.async_copy(ib_hbm.at[idx_v], ib_v, sem_bias)

    def fire(k):
        p = k % 2
        cu = pltpu.async_copy(
            w_hbm.at[idx_u.at[pl.ds(k * CHUNK, CHUNK)]], ubufs[p], sems[p])
        cv = pltpu.async_copy(
            w_hbm.at[idx_v.at[pl.ds(k * CHUNK, CHUNK)]], vbufs[p], sems[p])
        return cu, cv

    pending = fire(0)
    pltpu.sync_copy(gb_hbm, gb_v)
    cb1.wait()
    cb2.wait()

    gb = gb_v[...]
    lane = lax.iota(jnp.int32, L)

    for k in range(NCHUNK):
        nxt = fire(k + 1) if k + 1 < NCHUNK else None
        pending[0].wait()
        pending[1].wait()
        pending = nxt
        p = k % 2
        u_rows = ubufs[p]
        v_rows = vbufs[p]
        cbase = k * CHUNK

        def body(g, carry2, u_rows=u_rows, v_rows=v_rows, cbase=cbase):
            res = jnp.zeros((L,), jnp.float32)
            for j in range(L):
                r = g * L + j
                acc = None
                for c in range(EMBED // L):
                    uu = u_rows[r, pl.ds(c * L, L)]
                    vv = v_rows[r, pl.ds(EMBED + c * L, L)]
                    acc = uu * vv if acc is None else acc + uu * vv
                s = acc
                for kk in (8, 4, 2, 1):
                    s = s + _shuffle(s, lane ^ kk)
                res = jnp.where(lane == j, s, res)
            off = cbase + g * L
            pred16 = res + ub_v[pl.ds(off, L)] + ib_v[pl.ds(off, L)] + gb
            pred_v[pl.ds(off, L)] = pred16
            ctr_v[pl.ds(off, L)] = 1.0 / (1.0 + jnp.exp(-pred16))
            return carry2

        lax.fori_loop(0, CHUNK // L, body, 0)

    pltpu.sync_copy(pred_v, pred_hbm.at[pl.ds(base, B_PER_W)])
    pltpu.sync_copy(ctr_v, ctr_hbm.at[pl.ds(base, B_PER_W)])


_fm_call = pl.kernel(
    _fm_body,
    out_type=(
        jax.ShapeDtypeStruct((BATCH,), jnp.float32),
        jax.ShapeDtypeStruct((BATCH,), jnp.float32),
    ),
    mesh=plsc.VectorSubcoreMesh(
        core_axis_name="c", subcore_axis_name="s",
        num_cores=NC, num_subcores=NS,
    ),
    scratch_types=[
        pltpu.VMEM((B_PER_W,), jnp.int32),
        pltpu.VMEM((B_PER_W,), jnp.int32),
        pltpu.VMEM((CHUNK, PACK), jnp.float32),
        pltpu.VMEM((CHUNK, PACK), jnp.float32),
        pltpu.VMEM((CHUNK, PACK), jnp.float32),
        pltpu.VMEM((CHUNK, PACK), jnp.float32),
        pltpu.VMEM((B_PER_W,), jnp.float32),
        pltpu.VMEM((B_PER_W,), jnp.float32),
        pltpu.VMEM((L,), jnp.float32),
        pltpu.VMEM((B_PER_W,), jnp.float32),
        pltpu.VMEM((B_PER_W,), jnp.float32),
        pltpu.SemaphoreType.DMA,
        pltpu.SemaphoreType.DMA,
        pltpu.SemaphoreType.DMA,
    ],
)


@jax.jit
def kernel(user_id, item_id, user_factors, item_factors, user_bias,
           item_bias, global_bias):
    fused = _tp_call(user_factors.T, item_factors.T)
    gb16 = jnp.broadcast_to(global_bias.astype(jnp.float32), (L,))
    pred, ctr = _fm_call(user_id, item_id, fused,
                         user_bias, item_bias, gb16)
    return pred, ctr
